# baseline (speedup 1.0000x reference)
import math

import jax
import jax.numpy as jnp
from jax.experimental import pallas as pl
from jax.experimental.pallas import tpu as pltpu

_LEAKY_SLOPE = 0.2
_EPS = 1e-8
_HH = _WW = 4
_HW = _HH * _WW


def _leaky(v):
    return jnp.maximum(v, _LEAKY_SLOPE * v)


def _fused_kernel(x_ref, w1_ref, b1_ref, w2_ref, b2_ref, wrgb_ref, brgb_ref,
                  o_ref):
    c = x_ref.shape[1]
    x = x_ref[...]

    s2 = math.sqrt(2.0 / (9.0 * c))
    blocks = []
    for q in range(_HW):
        t = 15 - q
        hq = jnp.dot(x, w1_ref[:, t * c:(t + 1) * c],
                     preferred_element_type=jnp.float32)
        hq = _leaky(hq + b1_ref[...])
        ns = jax.lax.rsqrt(jnp.mean(hq * hq, axis=1, keepdims=True)
                           + _EPS) * s2
        blocks.append((hq * ns).astype(jnp.bfloat16))
    hn = jnp.concatenate(blocks, axis=1)

    for p in range(_HW):
        i, j = p // _WW, p % _WW
        j0, j1 = max(j - 1, 0), min(j + 1, 3)
        kw = (j1 - j0 + 1) * c
        acc = None
        for ki in range(3):
            ii = i + ki - 1
            if 0 <= ii < _HH:
                q0 = ii * _WW + j0
                t0 = ki * 3 + (j0 - j + 1)
                d = jnp.dot(hn[:, q0 * c:q0 * c + kw],
                            w2_ref[t0 * c:t0 * c + kw, :],
                            preferred_element_type=jnp.float32)
                acc = d if acc is None else acc + d
        z = _leaky(acc + b2_ref[...])
        ns = jax.lax.rsqrt(jnp.mean(z * z, axis=1, keepdims=True) + _EPS)
        rgb = jnp.dot(z, wrgb_ref[...],
                      preferred_element_type=jnp.float32) * ns + brgb_ref[...]
        o_ref[p] = rgb


def kernel(x, w1, b1, w2, b2, wrgb, brgb):
    n, c, h_in, w_in = x.shape
    assert h_in == 1 and w_in == 1
    c_out = w1.shape[0]

    s1 = math.sqrt(2.0 / (c * 4 * 4))
    s3 = math.sqrt(2.0 / (c_out * 1 * 1))

    w1_pm = jnp.transpose(
        w1.reshape(c_out, c, _HW),
        (1, 2, 0)).reshape(c, _HW * c_out).astype(jnp.bfloat16)
    b1_row = b1.reshape(1, c_out)

    w2_pm = jnp.transpose(w2.astype(jnp.bfloat16), (2, 3, 1, 0)).reshape(
        9 * c_out, c_out)
    b2_row = b2.reshape(1, c_out)

    wrgb_t = (wrgb[:, :, 0, 0] * s3).T
    brgb_row = brgb.reshape(1, 3)

    x2d = (x.reshape(n, c) * s1).astype(jnp.bfloat16)

    for cand in (256, 128, 64, 32, 16, 8):
        if n % cand == 0:
            blk = cand
            break
    else:
        blk = n

    out = pl.pallas_call(
        _fused_kernel,
        out_shape=jax.ShapeDtypeStruct((_HW, n, 3), jnp.float32),
        grid=(n // blk,),
        in_specs=[
            pl.BlockSpec((blk, c), lambda i: (i, 0)),
            pl.BlockSpec((c, _HW * c_out), lambda i: (0, 0)),
            pl.BlockSpec((1, c_out), lambda i: (0, 0)),
            pl.BlockSpec((9 * c_out, c_out), lambda i: (0, 0)),
            pl.BlockSpec((1, c_out), lambda i: (0, 0)),
            pl.BlockSpec((c_out, 3), lambda i: (0, 0)),
            pl.BlockSpec((1, 3), lambda i: (0, 0)),
        ],
        out_specs=pl.BlockSpec((_HW, blk, 3), lambda i: (0, i, 0)),
        compiler_params=pltpu.CompilerParams(
            dimension_semantics=("parallel",)),
    )(x2d, w1_pm, b1_row, w2_pm, b2_row, wrgb_t, brgb_row)

    return jnp.transpose(out, (1, 2, 0)).reshape(n, 3, _HH, _WW)


# speedup vs baseline: 1.0332x; 1.0332x over previous
import math

import jax
import jax.numpy as jnp
from jax.experimental import pallas as pl
from jax.experimental.pallas import tpu as pltpu

_LEAKY_SLOPE = 0.2
_EPS = 1e-8
_HH = _WW = 4
_HW = _HH * _WW


def _leaky(v):
    return jnp.maximum(v, _LEAKY_SLOPE * v)


def _fused_kernel(x_ref, w1_ref, b1_ref, w2_ref, b2_ref, wrgb_ref, brgb_ref,
                  o_ref):
    c = x_ref.shape[1]
    x = x_ref[...]

    s2 = math.sqrt(2.0 / (9.0 * c))
    h_all = jnp.dot(x, w1_ref[...], preferred_element_type=jnp.float32)
    blocks = []
    for q in range(_HW):
        hq = _leaky(h_all[:, (15 - q) * c:(16 - q) * c] + b1_ref[...])
        ns = jax.lax.rsqrt(jnp.mean(hq * hq, axis=1, keepdims=True)
                           + _EPS) * s2
        blocks.append((hq * ns).astype(jnp.bfloat16))
    hn = jnp.concatenate(blocks, axis=1)

    for p in range(_HW):
        i, j = p // _WW, p % _WW
        j0, j1 = max(j - 1, 0), min(j + 1, 3)
        kw = (j1 - j0 + 1) * c
        acc = None
        for ki in range(3):
            ii = i + ki - 1
            if 0 <= ii < _HH:
                q0 = ii * _WW + j0
                t0 = ki * 3 + (j0 - j + 1)
                d = jnp.dot(hn[:, q0 * c:q0 * c + kw],
                            w2_ref[t0 * c:t0 * c + kw, :],
                            preferred_element_type=jnp.float32)
                acc = d if acc is None else acc + d
        z = _leaky(acc + b2_ref[...])
        ns = jax.lax.rsqrt(jnp.mean(z * z, axis=1, keepdims=True) + _EPS)
        rgb = jnp.dot(z, wrgb_ref[...],
                      preferred_element_type=jnp.float32) * ns + brgb_ref[...]
        o_ref[p] = rgb


def kernel(x, w1, b1, w2, b2, wrgb, brgb):
    n, c, h_in, w_in = x.shape
    assert h_in == 1 and w_in == 1
    c_out = w1.shape[0]

    s1 = math.sqrt(2.0 / (c * 4 * 4))
    s3 = math.sqrt(2.0 / (c_out * 1 * 1))

    w1_pm = jnp.transpose(
        w1.reshape(c_out, c, _HW),
        (1, 2, 0)).reshape(c, _HW * c_out).astype(jnp.bfloat16)
    b1_row = b1.reshape(1, c_out)

    w2_pm = jnp.transpose(w2.astype(jnp.bfloat16), (2, 3, 1, 0)).reshape(
        9 * c_out, c_out)
    b2_row = b2.reshape(1, c_out)

    wrgb_t = (wrgb[:, :, 0, 0] * s3).T
    brgb_row = brgb.reshape(1, 3)

    x2d = (x.reshape(n, c) * s1).astype(jnp.bfloat16)

    for cand in (512, 256, 128, 64, 32, 16, 8):
        if n % cand == 0:
            blk = cand
            break
    else:
        blk = n

    out = pl.pallas_call(
        _fused_kernel,
        out_shape=jax.ShapeDtypeStruct((_HW, n, 3), jnp.float32),
        grid=(n // blk,),
        in_specs=[
            pl.BlockSpec((blk, c), lambda i: (i, 0)),
            pl.BlockSpec((c, _HW * c_out), lambda i: (0, 0)),
            pl.BlockSpec((1, c_out), lambda i: (0, 0)),
            pl.BlockSpec((9 * c_out, c_out), lambda i: (0, 0)),
            pl.BlockSpec((1, c_out), lambda i: (0, 0)),
            pl.BlockSpec((c_out, 3), lambda i: (0, 0)),
            pl.BlockSpec((1, 3), lambda i: (0, 0)),
        ],
        out_specs=pl.BlockSpec((_HW, blk, 3), lambda i: (0, i, 0)),
        compiler_params=pltpu.CompilerParams(
            dimension_semantics=("parallel",)),
    )(x2d, w1_pm, b1_row, w2_pm, b2_row, wrgb_t, brgb_row)

    return jnp.transpose(out, (1, 2, 0)).reshape(n, 3, _HH, _WW)
